# baseline (speedup 1.0000x reference)
import functools

import jax
import jax.numpy as jnp
from jax import lax
from jax.experimental import pallas as pl
from jax.experimental.pallas import tpu as pltpu
from jax.experimental.pallas import tpu_sc as plsc


_ROWS_PER_TILE = 256
_K_CHUNK = 1024


def _vq_body(x_ref, cbt_ref, idx_ref, loss_ref, c2_ref):
    i = pl.program_id(0)
    k_total = cbt_ref.shape[1]
    rows = x_ref.shape[0]

    @pl.when(i == 0)
    def _init():
        cb = cbt_ref[...]
        c2_ref[...] = jnp.sum(cb * cb, axis=0, keepdims=True)
        loss_ref[0, 0] = 0.0

    x = x_ref[...]
    x2 = jnp.sum(x * x, axis=1, keepdims=True)

    def kstep(k, carry):
        best_val, best_idx = carry
        kc = _K_CHUNK
        xc = jnp.dot(x, cbt_ref[:, pl.ds(k * kc, kc)],
                     preferred_element_type=jnp.float32)
        dist = x2 - 2.0 * xc + c2_ref[0:1, pl.ds(k * kc, kc)]
        cmin = jnp.min(dist, axis=1)
        iota = lax.broadcasted_iota(jnp.int32, dist.shape, 1)
        cidx = jnp.min(jnp.where(dist == cmin[:, None], iota, k_total),
                       axis=1) + k * kc
        upd = cmin < best_val
        return (jnp.where(upd, cmin, best_val),
                jnp.where(upd, cidx, best_idx))

    m0 = jnp.full((rows,), jnp.inf, jnp.float32)
    i0 = jnp.zeros((rows,), jnp.int32)
    best_val, best_idx = lax.fori_loop(0, k_total // _K_CHUNK, kstep, (m0, i0))
    idx_ref[...] = best_idx
    loss_ref[0, 0] += jnp.sum(best_val)


def _vq_pallas(x, cbt):
    r, d = x.shape
    k = cbt.shape[1]
    grid = (r // _ROWS_PER_TILE,)
    return pl.pallas_call(
        _vq_body,
        grid=grid,
        in_specs=[
            pl.BlockSpec((_ROWS_PER_TILE, d), lambda i: (i, 0)),
            pl.BlockSpec((d, k), lambda i: (0, 0)),
        ],
        out_specs=[
            pl.BlockSpec((_ROWS_PER_TILE,), lambda i: (i,)),
            pl.BlockSpec(memory_space=pltpu.SMEM),
        ],
        out_shape=[
            jax.ShapeDtypeStruct((r,), jnp.int32),
            jax.ShapeDtypeStruct((1, 1), jnp.float32),
        ],
        scratch_shapes=[pltpu.VMEM((1, k), jnp.float32)],
    )(x, cbt)


def _dec_body(cb_ref, w1_ref, b1_ref, w2_ref, b2_ref, out_ref):
    h = jnp.dot(cb_ref[...], w1_ref[...],
                preferred_element_type=jnp.float32) + b1_ref[...]
    h = jnp.maximum(h, 0.0)
    out_ref[...] = jnp.dot(h, w2_ref[...],
                           preferred_element_type=jnp.float32) + b2_ref[...]


def _decode_pallas(codebook, w1, b1, w2, b2):
    k, d = codebook.shape
    h = w1.shape[1]
    tile = 1024
    return pl.pallas_call(
        _dec_body,
        grid=(k // tile,),
        in_specs=[
            pl.BlockSpec((tile, d), lambda i: (i, 0)),
            pl.BlockSpec((d, h), lambda i: (0, 0)),
            pl.BlockSpec((1, h), lambda i: (0, 0)),
            pl.BlockSpec((h, d), lambda i: (0, 0)),
            pl.BlockSpec((1, d), lambda i: (0, 0)),
        ],
        out_specs=pl.BlockSpec((tile, d), lambda i: (i, 0)),
        out_shape=jax.ShapeDtypeStruct((k, d), jnp.float32),
    )(codebook, w1, b1, w2, b2)


_GATHER_CHUNK = 128


def _sc_gather(table, idx):
    r = idx.shape[0]
    d = table.shape[1]
    info = plsc.get_sparse_core_info()
    nw = info.num_cores * info.num_subcores
    b_per_w = r // nw
    chunks = b_per_w // _GATHER_CHUNK
    mesh = plsc.VectorSubcoreMesh(core_axis_name="c", subcore_axis_name="s")

    @functools.partial(
        pl.kernel,
        out_type=jax.ShapeDtypeStruct((r, d), jnp.float32),
        mesh=mesh,
        scratch_types=[
            pltpu.VMEM((_GATHER_CHUNK,), jnp.int32),
            pltpu.VMEM((_GATHER_CHUNK, d), jnp.float32),
            pltpu.SemaphoreType.DMA,
        ],
    )
    def gather_kernel(table_hbm, idx_hbm, out_hbm, idx_v, rows_v, sem):
        wid = lax.axis_index("s") * info.num_cores + lax.axis_index("c")
        base = wid * b_per_w

        def chunk(j, carry):
            o = base + j * _GATHER_CHUNK
            pltpu.sync_copy(idx_hbm.at[pl.ds(o, _GATHER_CHUNK)], idx_v)
            pltpu.async_copy(table_hbm.at[idx_v], rows_v, sem).wait()
            pltpu.sync_copy(rows_v, out_hbm.at[pl.ds(o, _GATHER_CHUNK)])
            return carry

        lax.fori_loop(0, chunks, chunk, 0)

    return gather_kernel(table, idx)


def kernel(features, codebook, W1, b1, W2, b2):
    b, n, d = features.shape
    r = b * n
    x = features.reshape(r, d)
    cbt = codebook.T
    idx_flat, minsum = _vq_pallas(x, cbt)
    decoded = _decode_pallas(codebook, W1, b1.reshape(1, -1), W2, b2.reshape(1, -1))
    recon = _sc_gather(decoded, idx_flat)
    commit_loss = 0.25 * minsum[0, 0] / (r * d)
    return recon.reshape(b, n, d), idx_flat.reshape(b, n), commit_loss


# speedup vs baseline: 1.7704x; 1.7704x over previous
import functools

import jax
import jax.numpy as jnp
from jax import lax
from jax.experimental import pallas as pl
from jax.experimental.pallas import tpu as pltpu
from jax.experimental.pallas import tpu_sc as plsc


_ROWS_PER_TILE = 256
_K_CHUNK = 1024


def _vq_body(x_ref, cbt2_ref, idx_ref, loss_ref, c2_ref):
    i = pl.program_id(0)
    k_total = cbt2_ref.shape[1]
    rows = x_ref.shape[0]
    kc = _K_CHUNK

    @pl.when(i == 0)
    def _init():
        cb = cbt2_ref[...]
        c2_ref[...] = 0.25 * jnp.sum(cb * cb, axis=0, keepdims=True)
        loss_ref[0, 0] = 0.0

    x = x_ref[...]
    x2 = jnp.sum(x * x, axis=1, keepdims=True)
    iota_f = lax.broadcasted_iota(jnp.int32, (rows, kc), 1).astype(jnp.float32)

    best_val = None
    for k in range(k_total // kc):
        xc2 = jnp.dot(x, cbt2_ref[:, k * kc:(k + 1) * kc],
                      preferred_element_type=jnp.float32)
        dist = x2 + xc2 + c2_ref[0:1, k * kc:(k + 1) * kc]
        cmin = jnp.min(dist, axis=1)
        cidx = jnp.min(jnp.where(dist == cmin[:, None], iota_f, jnp.inf),
                       axis=1) + float(k * kc)
        if best_val is None:
            best_val, best_idx = cmin, cidx
        else:
            upd = cmin < best_val
            best_val = jnp.where(upd, cmin, best_val)
            best_idx = jnp.where(upd, cidx, best_idx)

    idx_ref[...] = best_idx.astype(jnp.int32)
    loss_ref[0, 0] += jnp.sum(best_val)


def _vq_pallas(x, cbt2):
    r, d = x.shape
    k = cbt2.shape[1]
    grid = (r // _ROWS_PER_TILE,)
    return pl.pallas_call(
        _vq_body,
        grid=grid,
        in_specs=[
            pl.BlockSpec((_ROWS_PER_TILE, d), lambda i: (i, 0)),
            pl.BlockSpec((d, k), lambda i: (0, 0)),
        ],
        out_specs=[
            pl.BlockSpec((_ROWS_PER_TILE,), lambda i: (i,)),
            pl.BlockSpec(memory_space=pltpu.SMEM),
        ],
        out_shape=[
            jax.ShapeDtypeStruct((r,), jnp.int32),
            jax.ShapeDtypeStruct((1, 1), jnp.float32),
        ],
        scratch_shapes=[pltpu.VMEM((1, k), jnp.float32)],
    )(x, cbt2)


def _dec_body(cb_ref, w1_ref, b1_ref, w2_ref, b2_ref, out_ref):
    h = jnp.dot(cb_ref[...], w1_ref[...],
                preferred_element_type=jnp.float32) + b1_ref[...]
    h = jnp.maximum(h, 0.0)
    out_ref[...] = jnp.dot(h, w2_ref[...],
                           preferred_element_type=jnp.float32) + b2_ref[...]


def _decode_pallas(codebook, w1, b1, w2, b2):
    k, d = codebook.shape
    h = w1.shape[1]
    tile = 1024
    return pl.pallas_call(
        _dec_body,
        grid=(k // tile,),
        in_specs=[
            pl.BlockSpec((tile, d), lambda i: (i, 0)),
            pl.BlockSpec((d, h), lambda i: (0, 0)),
            pl.BlockSpec((1, h), lambda i: (0, 0)),
            pl.BlockSpec((h, d), lambda i: (0, 0)),
            pl.BlockSpec((1, d), lambda i: (0, 0)),
        ],
        out_specs=pl.BlockSpec((tile, d), lambda i: (i, 0)),
        out_shape=jax.ShapeDtypeStruct((k, d), jnp.float32),
    )(codebook, w1, b1, w2, b2)


_GATHER_CHUNK = 128


def _sc_gather(table, idx):
    r = idx.shape[0]
    d = table.shape[1]
    info = plsc.get_sparse_core_info()
    nw = info.num_cores * info.num_subcores
    b_per_w = r // nw
    chunks = b_per_w // _GATHER_CHUNK
    mesh = plsc.VectorSubcoreMesh(core_axis_name="c", subcore_axis_name="s")

    @functools.partial(
        pl.kernel,
        out_type=jax.ShapeDtypeStruct((r, d), jnp.float32),
        mesh=mesh,
        scratch_types=[
            pltpu.VMEM((_GATHER_CHUNK,), jnp.int32),
            pltpu.VMEM((_GATHER_CHUNK, d), jnp.float32),
            pltpu.SemaphoreType.DMA,
        ],
    )
    def gather_kernel(table_hbm, idx_hbm, out_hbm, idx_v, rows_v, sem):
        wid = lax.axis_index("s") * info.num_cores + lax.axis_index("c")
        base = wid * b_per_w

        def chunk(j, carry):
            o = base + j * _GATHER_CHUNK
            pltpu.sync_copy(idx_hbm.at[pl.ds(o, _GATHER_CHUNK)], idx_v)
            pltpu.async_copy(table_hbm.at[idx_v], rows_v, sem).wait()
            pltpu.sync_copy(rows_v, out_hbm.at[pl.ds(o, _GATHER_CHUNK)])
            return carry

        lax.fori_loop(0, chunks, chunk, 0)

    return gather_kernel(table, idx)


def kernel(features, codebook, W1, b1, W2, b2):
    b, n, d = features.shape
    r = b * n
    x = features.reshape(r, d)
    cbt2 = -2.0 * codebook.T
    idx_flat, minsum = _vq_pallas(x, cbt2)
    decoded = _decode_pallas(codebook, W1, b1.reshape(1, -1), W2, b2.reshape(1, -1))
    recon = _sc_gather(decoded, idx_flat)
    commit_loss = 0.25 * minsum[0, 0] / (r * d)
    return recon.reshape(b, n, d), idx_flat.reshape(b, n), commit_loss
